# baseline (speedup 1.0000x reference)
import functools
import math

import jax
import jax.numpy as jnp
from jax import lax
from jax.experimental import pallas as pl
from jax.experimental.pallas import tpu as pltpu
from jax.experimental.pallas import tpu_sc as plsc

N0 = 10000
E0 = 320000
D = 128
H = 128
O = 128

LANES = 16
NP = 10240
C = 128
HC = H // 2
NT = 16
NCH = 160
EP = NCH * NT * C
NBUF = 4
BLK = 8
NBLK = NCH // BLK
QB = 4
RPT = NP // LANES
BR = 256


def _dense_body(x_ref, w_ref, att_ref, h2_ref, ab_ref):
    h = jnp.dot(x_ref[...], w_ref[...], preferred_element_type=jnp.float32)
    h2_ref[0] = h[:, :HC]
    h2_ref[1] = h[:, HC:]
    ab_ref[...] = jnp.dot(h, att_ref[...], preferred_element_type=jnp.float32)


def _dense1(x_p, W, att_mat):
    return pl.pallas_call(
        _dense_body,
        grid=(NP // BR,),
        in_specs=[
            pl.BlockSpec((BR, D), lambda i: (i, 0)),
            pl.BlockSpec((D, H), lambda i: (0, 0)),
            pl.BlockSpec((H, 2), lambda i: (0, 0)),
        ],
        out_specs=[
            pl.BlockSpec((2, BR, HC), lambda i: (0, i, 0)),
            pl.BlockSpec((BR, 2), lambda i: (i, 0)),
        ],
        out_shape=[
            jax.ShapeDtypeStruct((2, NP, HC), jnp.float32),
            jax.ShapeDtypeStruct((NP, 2), jnp.float32),
        ],
    )(x_p, W, att_mat)


def _sc_body(edges_hbm, ab_hbm, h3_hbm, acc_out, den_out,
             ab_v, siB, diB, wvB, wz, rows4, accS, denS,
             sg0, sg1, sg2, sg3, ss0, ss1, ss2, ss3, smi):
    cid = lax.axis_index("c")
    sid = lax.axis_index("s")
    semg = [sg0, sg1, sg2, sg3]
    sems = [ss0, ss1, ss2, ss3]
    coff = cid * NP

    pltpu.sync_copy(ab_hbm, ab_v)

    zeros16 = jnp.zeros((LANES,), jnp.float32)

    def _zrow(j, carry):
        for p in range(NBUF):
            for k in range(HC // LANES):
                rows4[p, j, pl.ds(k * LANES, LANES)] = zeros16
        return carry

    lax.fori_loop(0, C, _zrow, 0)
    for k in range(C // LANES):
        wz[pl.ds(k * LANES, LANES)] = zeros16

    r0 = sid * RPT
    for q in range(RPT // C):
        pltpu.sync_copy(rows4.at[0], accS.at[pl.ds(r0 + q * C, C), :])
        pltpu.sync_copy(wz, denS.at[pl.ds(r0 + q * C, C)])
    plsc.subcore_barrier()

    def _wblock(qdst, gsrc):
        @plsc.parallel_loop(0, BLK, 1)
        def _wrow(j8):
            for j in range(C // LANES):
                sl = pl.ds(j * LANES, LANES)
                s16 = siB[qdst, j8, sl]
                d16 = diB[qdst, j8, sl]
                e = (plsc.load_gather(ab_v, [s16 * 2])
                     + plsc.load_gather(ab_v, [d16 * 2 + 1]))
                e = jnp.where(e < 0.0, e * 0.2, e)
                wvB[qdst, j8, sl] = jnp.exp(e)
                siB[qdst, j8, sl] = s16 + coff

    pltpu.sync_copy(edges_hbm.at[0, sid, 0], siB.at[0])
    pltpu.sync_copy(edges_hbm.at[1, sid, 0], diB.at[0])
    _wblock(0, 0)
    pltpu.async_copy(edges_hbm.at[0, sid, 1], siB.at[1], smi)
    pltpu.async_copy(edges_hbm.at[1, sid, 1], diB.at[1], smi)
    for p in range(2, NBUF):
        pltpu.async_copy(rows4.at[p], accS.at[diB.at[0, p]], sems[p], add=True)
        pltpu.async_copy(wz, denS.at[diB.at[0, p]], sems[p], add=True)
    pltpu.async_copy(h3_hbm.at[siB.at[0, 0]], rows4.at[0], semg[0])
    pltpu.async_copy(h3_hbm.at[siB.at[0, 1]], rows4.at[1], semg[1])

    def _block(g, carry):
        q = lax.rem(g, QB)
        qn1 = lax.rem(g + 1, QB)
        qn2 = lax.rem(g + 2, QB)
        gn1 = jnp.minimum(g + 1, NBLK - 1)
        gn2 = jnp.minimum(g + 2, NBLK - 1)
        pltpu.make_async_copy(edges_hbm.at[0, sid, gn1], siB.at[qn1],
                              smi).wait()
        pltpu.make_async_copy(edges_hbm.at[1, sid, gn1], diB.at[qn1],
                              smi).wait()
        _wblock(qn1, gn1)
        pltpu.async_copy(edges_hbm.at[0, sid, gn2], siB.at[qn2], smi)
        pltpu.async_copy(edges_hbm.at[1, sid, gn2], diB.at[qn2], smi)

        for j in range(BLK):
            p = j % NBUF
            pn = (p + 2) % NBUF
            qb, jb = (q, j + 2) if j < BLK - 2 else (qn1, j - (BLK - 2))
            pltpu.make_async_copy(rows4.at[pn], accS.at[diB.at[q, 0]],
                                  sems[pn]).wait()
            pltpu.make_async_copy(wz, denS.at[diB.at[q, 0]], sems[pn]).wait()
            pltpu.async_copy(h3_hbm.at[siB.at[qb, jb]], rows4.at[pn],
                             semg[pn])
            pltpu.make_async_copy(h3_hbm.at[siB.at[q, j]], rows4.at[p],
                                  semg[p]).wait()

            @plsc.parallel_loop(0, C // LANES, 1, unroll=4)
            def _scale(gg, _p=p, _j=j):
                w16 = wvB[q, _j, pl.ds(gg * LANES, LANES)]
                for t in range(LANES):
                    w = w16[t]
                    for k in range(HC // LANES):
                        sl = pl.ds(k * LANES, LANES)
                        rows4[_p, gg * LANES + t, sl] = (
                            rows4[_p, gg * LANES + t, sl] * w)
            pltpu.async_copy(rows4.at[p], accS.at[diB.at[q, j]], sems[p],
                             add=True)
            pltpu.async_copy(wvB.at[q, j], denS.at[diB.at[q, j]], sems[p],
                             add=True)
        return carry

    lax.fori_loop(0, NBLK, _block, 0)

    qtail = (NBLK + 1) % QB
    pltpu.make_async_copy(edges_hbm.at[0, sid, 0], siB.at[qtail], smi).wait()
    pltpu.make_async_copy(edges_hbm.at[1, sid, 0], diB.at[qtail], smi).wait()
    pltpu.make_async_copy(h3_hbm.at[siB.at[0, 0]], rows4.at[0], semg[0]).wait()
    pltpu.make_async_copy(h3_hbm.at[siB.at[0, 0]], rows4.at[1], semg[1]).wait()
    for p in range(2, NBUF):
        pltpu.make_async_copy(rows4.at[p], accS.at[diB.at[0, 0]],
                              sems[p]).wait()
        pltpu.make_async_copy(wz, denS.at[diB.at[0, 0]], sems[p]).wait()
    plsc.subcore_barrier()

    pltpu.sync_copy(accS.at[pl.ds(r0, RPT), :], acc_out.at[cid, pl.ds(r0, RPT), :])
    pltpu.sync_copy(denS.at[pl.ds(r0, RPT)], den_out.at[cid, pl.ds(r0, RPT)])


def _sc_aggregate(edges, ab_flat, h):
    mesh = plsc.VectorSubcoreMesh(core_axis_name="c", subcore_axis_name="s")
    kern = functools.partial(
        pl.kernel,
        mesh=mesh,
        compiler_params=pltpu.CompilerParams(
            needs_layout_passes=False, use_tc_tiling_on_sc=False),
        out_type=[
            jax.ShapeDtypeStruct((2, NP, HC), jnp.float32),
            jax.ShapeDtypeStruct((2, NP), jnp.float32),
        ],
        scratch_types=[
            pltpu.VMEM((2 * NP,), jnp.float32),
            pltpu.VMEM((QB, BLK, C), jnp.int32),
            pltpu.VMEM((QB, BLK, C), jnp.int32),
            pltpu.VMEM((QB, BLK, C), jnp.float32),
            pltpu.VMEM((C,), jnp.float32),
            pltpu.VMEM((NBUF, C, HC), jnp.float32),
            pltpu.VMEM_SHARED((NP, HC), jnp.float32),
            pltpu.VMEM_SHARED((NP,), jnp.float32),
        ] + [pltpu.SemaphoreType.DMA] * (2 * NBUF + 1),
    )(_sc_body)
    return kern(edges, ab_flat, h)


_BN_SCALE = 1.0 / math.sqrt(1.0 + 1e-5)


def _epi_body(acc_ref, den_ref, h_ref, ab_ref, par_ref, fcw_ref, o_ref):
    i = pl.program_id(0)
    den2 = den_ref[:, pl.ds(i * BR, BR)]
    ab = ab_ref[...]
    e = ab[:, 0] + ab[:, 1]
    e = jnp.where(e < 0.0, e * 0.2, e)
    ws = jnp.exp(e)
    acc = jnp.concatenate([acc_ref[0], acc_ref[1]], axis=1)
    hh = jnp.concatenate([h_ref[0], h_ref[1]], axis=1)
    acc = acc + ws[:, None] * hh
    den = (den2[0] + den2[1]) * 0.5 + ws
    node = acc / (den + 1e-16)[:, None]
    node = node + par_ref[0:1, :]
    node = node * _BN_SCALE * par_ref[1:2, :] + par_ref[2:3, :]
    node = jnp.maximum(node, 0.0)
    o_ref[...] = jnp.dot(node, fcw_ref[...],
                         preferred_element_type=jnp.float32) + par_ref[3:4, :]


def _epilogue(accp, denp, h, ab, params, fc_W):
    return pl.pallas_call(
        _epi_body,
        grid=(NP // BR,),
        in_specs=[
            pl.BlockSpec((2, BR, HC), lambda i: (0, i, 0)),
            pl.BlockSpec((2, NP), lambda i: (0, 0)),
            pl.BlockSpec((2, BR, HC), lambda i: (0, i, 0)),
            pl.BlockSpec((BR, 2), lambda i: (i, 0)),
            pl.BlockSpec((4, H), lambda i: (0, 0)),
            pl.BlockSpec((H, O), lambda i: (0, 0)),
        ],
        out_specs=pl.BlockSpec((BR, O), lambda i: (i, 0)),
        out_shape=jax.ShapeDtypeStruct((N0, O), jnp.float32),
    )(accp, denp, h, ab, params, fc_W)


def kernel(x, edge_index, W, att_src, att_dst, bias, bn_gamma, bn_beta,
           fc_W, fc_b):
    pad = EP - E0
    pad_idx = N0 + (jnp.arange(pad, dtype=jnp.int32) % (NP - N0))
    edges = jnp.concatenate(
        [edge_index.astype(jnp.int32), jnp.broadcast_to(pad_idx, (2, pad))],
        axis=1).reshape(2, NT, NBLK, BLK, C)
    att_mat = jnp.stack([att_src, att_dst], axis=1)
    params = jnp.stack([bias, bn_gamma, bn_beta, fc_b])

    h2, ab = _dense1(x, W, att_mat)
    ab_flat = ab.reshape(2 * NP)
    h3 = h2.reshape(2 * NP, HC)
    accp, denp = _sc_aggregate(edges, ab_flat, h3)
    return _epilogue(accp, denp, h2, ab, params, fc_W)


# speedup vs baseline: 1.0646x; 1.0646x over previous
import functools
import math

import jax
import jax.numpy as jnp
from jax import lax
from jax.experimental import pallas as pl
from jax.experimental.pallas import tpu as pltpu
from jax.experimental.pallas import tpu_sc as plsc

N0 = 10000
E0 = 320000
D = 128
H = 128
O = 128

LANES = 16
NP = 10240
C = 128
HC = H // 2
NT = 16
NCH = 160
EP = NCH * NT * C
NBUF = 4
BLK = 8
NBLK = NCH // BLK
QB = 4
RPT = NP // LANES
BR = 256


def _dense_body(x_ref, w_ref, att_ref, h2_ref, ab_ref):
    h = jnp.dot(x_ref[...], w_ref[...], preferred_element_type=jnp.float32)
    h2_ref[0] = h[:, :HC]
    h2_ref[1] = h[:, HC:]
    ab_ref[...] = jnp.dot(h, att_ref[...], preferred_element_type=jnp.float32)


def _dense1(x_p, W, att_mat):
    return pl.pallas_call(
        _dense_body,
        grid=(NP // BR,),
        in_specs=[
            pl.BlockSpec((BR, D), lambda i: (i, 0)),
            pl.BlockSpec((D, H), lambda i: (0, 0)),
            pl.BlockSpec((H, 2), lambda i: (0, 0)),
        ],
        out_specs=[
            pl.BlockSpec((2, BR, HC), lambda i: (0, i, 0)),
            pl.BlockSpec((BR, 2), lambda i: (i, 0)),
        ],
        out_shape=[
            jax.ShapeDtypeStruct((2, NP, HC), jnp.float32),
            jax.ShapeDtypeStruct((NP, 2), jnp.float32),
        ],
    )(x_p, W, att_mat)


def _sc_body(edges_hbm, ab_hbm, h3_hbm, acc_out, den_out,
             ab_v, siB, diB, wvB, wz, rows4, accS, denS,
             sg0, sg1, sg2, sg3, ss0, ss1, ss2, ss3, smi):
    cid = lax.axis_index("c")
    sid = lax.axis_index("s")
    semg = [sg0, sg1, sg2, sg3]
    sems = [ss0, ss1, ss2, ss3]
    coff = cid * NP

    pltpu.sync_copy(ab_hbm, ab_v)

    zeros16 = jnp.zeros((LANES,), jnp.float32)

    def _zrow(j, carry):
        for p in range(NBUF):
            for k in range(HC // LANES):
                rows4[p, j, pl.ds(k * LANES, LANES)] = zeros16
        return carry

    lax.fori_loop(0, C, _zrow, 0)
    for k in range(C // LANES):
        wz[pl.ds(k * LANES, LANES)] = zeros16

    r0 = sid * RPT
    for q in range(RPT // C):
        pltpu.sync_copy(rows4.at[0], accS.at[pl.ds(r0 + q * C, C), :])
        pltpu.sync_copy(wz, denS.at[pl.ds(r0 + q * C, C)])
    plsc.subcore_barrier()

    def _wblock(qdst, gsrc):
        @plsc.parallel_loop(0, BLK, 1)
        def _wrow(j8):
            for j in range(C // LANES):
                sl = pl.ds(j * LANES, LANES)
                s16 = siB[qdst, j8, sl]
                d16 = diB[qdst, j8, sl]
                e = (plsc.load_gather(ab_v, [s16 * 2])
                     + plsc.load_gather(ab_v, [d16 * 2 + 1]))
                e = jnp.where(e < 0.0, e * 0.2, e)
                wvB[qdst, j8, sl] = jnp.exp(e)
                siB[qdst, j8, sl] = s16 + coff

    pltpu.sync_copy(edges_hbm.at[0, sid, 0], siB.at[0])
    pltpu.sync_copy(edges_hbm.at[1, sid, 0], diB.at[0])
    _wblock(0, 0)
    pltpu.async_copy(edges_hbm.at[0, sid, 1], siB.at[1], smi)
    pltpu.async_copy(edges_hbm.at[1, sid, 1], diB.at[1], smi)
    for p in range(2, NBUF):
        pltpu.async_copy(rows4.at[p], accS.at[diB.at[0, p]], sems[p], add=True)
        pltpu.async_copy(wz, denS.at[diB.at[0, p]], sems[p], add=True)
    pltpu.async_copy(h3_hbm.at[siB.at[0, 0]], rows4.at[0], semg[0])
    pltpu.async_copy(h3_hbm.at[siB.at[0, 1]], rows4.at[1], semg[1])

    def _block(g, carry):
        q = lax.rem(g, QB)
        qn1 = lax.rem(g + 1, QB)
        qn2 = lax.rem(g + 2, QB)
        gn1 = jnp.minimum(g + 1, NBLK - 1)
        gn2 = jnp.minimum(g + 2, NBLK - 1)
        pltpu.make_async_copy(edges_hbm.at[0, sid, gn1], siB.at[qn1],
                              smi).wait()
        pltpu.make_async_copy(edges_hbm.at[1, sid, gn1], diB.at[qn1],
                              smi).wait()
        _wblock(qn1, gn1)
        pltpu.async_copy(edges_hbm.at[0, sid, gn2], siB.at[qn2], smi)
        pltpu.async_copy(edges_hbm.at[1, sid, gn2], diB.at[qn2], smi)

        for j in range(BLK):
            p = j % NBUF
            pn = (p + 2) % NBUF
            qb, jb = (q, j + 2) if j < BLK - 2 else (qn1, j - (BLK - 2))
            pltpu.make_async_copy(rows4.at[pn], accS.at[diB.at[q, 0]],
                                  sems[pn]).wait()
            pltpu.make_async_copy(wz, denS.at[diB.at[q, 0]], sems[pn]).wait()
            pltpu.async_copy(h3_hbm.at[siB.at[qb, jb]], rows4.at[pn],
                             semg[pn])
            pltpu.make_async_copy(h3_hbm.at[siB.at[q, j]], rows4.at[p],
                                  semg[p]).wait()

            @plsc.parallel_loop(0, C // LANES, 1, unroll=2)
            def _scale(gg, _p=p, _j=j):
                w16 = wvB[q, _j, pl.ds(gg * LANES, LANES)]
                for t in range(LANES):
                    w = w16[t]
                    for k in range(HC // LANES):
                        sl = pl.ds(k * LANES, LANES)
                        rows4[_p, gg * LANES + t, sl] = (
                            rows4[_p, gg * LANES + t, sl] * w)
            pltpu.async_copy(rows4.at[p], accS.at[diB.at[q, j]], sems[p],
                             add=True)
            pltpu.async_copy(wvB.at[q, j], denS.at[diB.at[q, j]], sems[p],
                             add=True)
        return carry

    lax.fori_loop(0, NBLK, _block, 0)

    qtail = (NBLK + 1) % QB
    pltpu.make_async_copy(edges_hbm.at[0, sid, 0], siB.at[qtail], smi).wait()
    pltpu.make_async_copy(edges_hbm.at[1, sid, 0], diB.at[qtail], smi).wait()
    pltpu.make_async_copy(h3_hbm.at[siB.at[0, 0]], rows4.at[0], semg[0]).wait()
    pltpu.make_async_copy(h3_hbm.at[siB.at[0, 0]], rows4.at[1], semg[1]).wait()
    for p in range(2, NBUF):
        pltpu.make_async_copy(rows4.at[p], accS.at[diB.at[0, 0]],
                              sems[p]).wait()
        pltpu.make_async_copy(wz, denS.at[diB.at[0, 0]], sems[p]).wait()
    plsc.subcore_barrier()

    pltpu.sync_copy(accS.at[pl.ds(r0, RPT), :], acc_out.at[cid, pl.ds(r0, RPT), :])
    pltpu.sync_copy(denS.at[pl.ds(r0, RPT)], den_out.at[cid, pl.ds(r0, RPT)])


def _sc_aggregate(edges, ab_flat, h):
    mesh = plsc.VectorSubcoreMesh(core_axis_name="c", subcore_axis_name="s")
    kern = functools.partial(
        pl.kernel,
        mesh=mesh,
        compiler_params=pltpu.CompilerParams(
            needs_layout_passes=False, use_tc_tiling_on_sc=False),
        out_type=[
            jax.ShapeDtypeStruct((2, NP, HC), jnp.float32),
            jax.ShapeDtypeStruct((2, NP), jnp.float32),
        ],
        scratch_types=[
            pltpu.VMEM((2 * NP,), jnp.float32),
            pltpu.VMEM((QB, BLK, C), jnp.int32),
            pltpu.VMEM((QB, BLK, C), jnp.int32),
            pltpu.VMEM((QB, BLK, C), jnp.float32),
            pltpu.VMEM((C,), jnp.float32),
            pltpu.VMEM((NBUF, C, HC), jnp.float32),
            pltpu.VMEM_SHARED((NP, HC), jnp.float32),
            pltpu.VMEM_SHARED((NP,), jnp.float32),
        ] + [pltpu.SemaphoreType.DMA] * (2 * NBUF + 1),
    )(_sc_body)
    return kern(edges, ab_flat, h)


_BN_SCALE = 1.0 / math.sqrt(1.0 + 1e-5)


def _epi_body(acc_ref, den_ref, h_ref, ab_ref, par_ref, fcw_ref, o_ref):
    i = pl.program_id(0)
    den2 = den_ref[:, pl.ds(i * BR, BR)]
    ab = ab_ref[...]
    e = ab[:, 0] + ab[:, 1]
    e = jnp.where(e < 0.0, e * 0.2, e)
    ws = jnp.exp(e)
    acc = jnp.concatenate([acc_ref[0], acc_ref[1]], axis=1)
    hh = jnp.concatenate([h_ref[0], h_ref[1]], axis=1)
    acc = acc + ws[:, None] * hh
    den = (den2[0] + den2[1]) * 0.5 + ws
    node = acc / (den + 1e-16)[:, None]
    node = node + par_ref[0:1, :]
    node = node * _BN_SCALE * par_ref[1:2, :] + par_ref[2:3, :]
    node = jnp.maximum(node, 0.0)
    o_ref[...] = jnp.dot(node, fcw_ref[...],
                         preferred_element_type=jnp.float32) + par_ref[3:4, :]


def _epilogue(accp, denp, h, ab, params, fc_W):
    return pl.pallas_call(
        _epi_body,
        grid=(NP // BR,),
        in_specs=[
            pl.BlockSpec((2, BR, HC), lambda i: (0, i, 0)),
            pl.BlockSpec((2, NP), lambda i: (0, 0)),
            pl.BlockSpec((2, BR, HC), lambda i: (0, i, 0)),
            pl.BlockSpec((BR, 2), lambda i: (i, 0)),
            pl.BlockSpec((4, H), lambda i: (0, 0)),
            pl.BlockSpec((H, O), lambda i: (0, 0)),
        ],
        out_specs=pl.BlockSpec((BR, O), lambda i: (i, 0)),
        out_shape=jax.ShapeDtypeStruct((N0, O), jnp.float32),
    )(accp, denp, h, ab, params, fc_W)


def kernel(x, edge_index, W, att_src, att_dst, bias, bn_gamma, bn_beta,
           fc_W, fc_b):
    pad = EP - E0
    pad_idx = N0 + (jnp.arange(pad, dtype=jnp.int32) % (NP - N0))
    edges = jnp.concatenate(
        [edge_index.astype(jnp.int32), jnp.broadcast_to(pad_idx, (2, pad))],
        axis=1).reshape(2, NT, NBLK, BLK, C)
    att_mat = jnp.stack([att_src, att_dst], axis=1)
    params = jnp.stack([bias, bn_gamma, bn_beta, fc_b])

    h2, ab = _dense1(x, W, att_mat)
    ab_flat = ab.reshape(2 * NP)
    h3 = h2.reshape(2 * NP, HC)
    accp, denp = _sc_aggregate(edges, ab_flat, h3)
    return _epilogue(accp, denp, h2, ab, params, fc_W)
